# Initial kernel scaffold; baseline (speedup 1.0000x reference)
import jax
import jax.numpy as jnp
from jax.experimental import pallas as pl


def kernel(x, edge_index, W1, att_src1, att_dst1, b1, g1, be1, W2, att_src2, att_dst2, b2, g2, be2):
    raise NotImplementedError("write your pallas kernel here")


# speedup vs baseline: 1.9751x; 1.9751x over previous
import functools

import jax
import jax.numpy as jnp
from jax import lax
from jax.experimental import pallas as pl
from jax.experimental.pallas import tpu as pltpu
from jax.experimental.pallas import tpu_sc as plsc

N = 10000
E = 160000
HID = 256
D = 512
NPAD = 10240
CHUNK = 160
NCHUNK = 64
NW = 32
NPASS = 2
EP = E + NCHUNK * 16 + 2048


def _make_agg():
    mesh = plsc.VectorSubcoreMesh(core_axis_name="c", subcore_axis_name="s")

    @functools.partial(
        pl.kernel,
        mesh=mesh,
        out_type=jax.ShapeDtypeStruct((NPAD, D), jnp.float32),
        scratch_types=[
            pltpu.VMEM((CHUNK, D), jnp.float32),
            pltpu.VMEM((NCHUNK * 8 + 16,), jnp.int32),
            pltpu.VMEM((2048,), jnp.int32),
            pltpu.VMEM((2048,), jnp.int32),
            pltpu.VMEM((2048,), jnp.float32),
            pltpu.VMEM((2048,), jnp.float32),
            pltpu.VMEM((16, D), jnp.float32),
            pltpu.VMEM((16, D), jnp.float32),
            pltpu.SemaphoreType.DMA,
            pltpu.SemaphoreType.DMA,
        ],
    )
    def agg(h_hbm, srcs_hbm, dls_hbm, w0_hbm, w1_hbm, starts_hbm, out_hbm,
            acc, sv, esrc, edl, ew0, ew1, stage0, stage1, sem0, sem1):
        wid = lax.axis_index("s") * 2 + lax.axis_index("c")
        zf = jnp.zeros((16,), jnp.float32)

        pltpu.sync_copy(starts_hbm, sv)

        for p in range(NPASS):
            c = p * NW + wid
            lo = pl.multiple_of(c * CHUNK, 32)

            def zrow(r, _):
                for j in range(32):
                    acc[r, pl.ds(j * 16, 16)] = zf
                return 0

            lax.fori_loop(0, CHUNK, zrow, 0)

            se = sv[pl.ds(pl.multiple_of(c * 8, 8), 16)]
            start = se[0]
            end = se[1]
            ngroups = (end - start) // 16

            stages = (stage0, stage1)
            sems = (sem0, sem1)

            nslabs = (ngroups + 127) // 128

            def slab(sb, _):
                gbase = sb * 128
                ng = jnp.minimum(ngroups - gbase, 128)
                ebase = pl.multiple_of(start + gbase * 16, 16)
                pltpu.sync_copy(srcs_hbm.at[pl.ds(ebase, 2048)], esrc)
                pltpu.sync_copy(dls_hbm.at[pl.ds(ebase, 2048)], edl)
                pltpu.sync_copy(w0_hbm.at[pl.ds(ebase, 2048)], ew0)
                pltpu.sync_copy(w1_hbm.at[pl.ds(ebase, 2048)], ew1)

                @pl.when(ng > 0)
                def _():
                    idx0 = esrc[pl.ds(0, 16)]
                    pltpu.make_async_copy(h_hbm.at[idx0], stage0, sem0).start()

                def pair(q, _):
                    for b in range(2):
                        g = q * 2 + b

                        @pl.when(g < ng)
                        def _():
                            idxg = esrc[pl.ds(g * 16, 16)]
                            pltpu.make_async_copy(
                                h_hbm.at[idxg], stages[b], sems[b]).wait()

                            @pl.when(g + 1 < ng)
                            def _():
                                idxn = esrc[pl.ds((g + 1) * 16, 16)]
                                pltpu.make_async_copy(
                                    h_hbm.at[idxn], stages[1 - b], sems[1 - b]).start()

                            dlv = edl[pl.ds(g * 16, 16)]
                            w0v = ew0[pl.ds(g * 16, 16)]
                            w1v = ew1[pl.ds(g * 16, 16)]
                            for k in range(16):
                                dl = dlv[k]
                                w0k = w0v[k]
                                w1k = w1v[k]

                                def jbody(jq, _, _k=k, _dl=dl, _w0=w0k, _w1=w1k, _b=b):
                                    off0 = pl.multiple_of(_k * 0 + jq * 64, 64)
                                    for u in range(4):
                                        off = off0 + u * 16
                                        w = jnp.where(jq * 64 + u * 16 < 256, _w0, _w1)
                                        acc[_dl, pl.ds(off, 16)] = (
                                            acc[_dl, pl.ds(off, 16)]
                                            + w * stages[_b][_k, pl.ds(off, 16)])
                                    return 0

                                lax.fori_loop(0, 8, jbody, 0)
                    return 0

                lax.fori_loop(0, (ng + 1) // 2, pair, 0)
                return 0

            lax.fori_loop(0, nslabs, slab, 0)
            pltpu.sync_copy(acc, out_hbm.at[pl.ds(lo, CHUNK)])

    return agg


def _edge_prep(src, dst):
    perm = jnp.argsort(dst, stable=True)
    dsts = dst[perm]
    srcs = src[perm]
    first = jnp.searchsorted(
        dsts, jnp.arange(NCHUNK + 1, dtype=jnp.int32) * CHUNK).astype(jnp.int32)
    cnt = first[1:] - first[:-1]
    pcnt = (cnt + 15) // 16 * 16
    sa = jnp.concatenate(
        [jnp.zeros((1,), jnp.int32), jnp.cumsum(pcnt).astype(jnp.int32)])
    ci = dsts // CHUNK
    pos = sa[ci] + (jnp.arange(E, dtype=jnp.int32) - first[ci])
    srcs_p = jnp.zeros((EP,), jnp.int32).at[pos].set(srcs, unique_indices=True)
    dls_p = jnp.zeros((EP,), jnp.int32).at[pos].set(
        dsts % CHUNK, unique_indices=True)
    idx8 = jnp.arange(NCHUNK, dtype=jnp.int32) * 8
    s8 = jnp.zeros((NCHUNK * 8 + 16,), jnp.int32)
    s8 = s8.at[idx8].set(sa[:-1]).at[idx8 + 1].set(sa[1:])
    return perm, pos, srcs_p, dls_p, s8


def _batch_norm(x, gamma, beta):
    mu = x.mean(axis=0, keepdims=True)
    var = x.var(axis=0, keepdims=True)
    return (x - mu) / jnp.sqrt(var + 1e-5) * gamma + beta


def kernel(x, edge_index, W1, att_src1, att_dst1, b1, g1, be1,
           W2, att_src2, att_dst2, b2, g2, be2):
    src = edge_index[0].astype(jnp.int32)
    dst = edge_index[1].astype(jnp.int32)

    perm, pos, srcs_p, dls_p, s8 = _edge_prep(src, dst)
    sc_agg = _make_agg()

    h = (x @ W1).reshape(N, 2, HID)
    alpha_s = jnp.sum(h * att_src1[None, :, :], axis=-1)
    alpha_d = jnp.sum(h * att_dst1[None, :, :], axis=-1)
    e = jax.nn.leaky_relu(alpha_s[src] + alpha_d[dst], negative_slope=0.2)
    m = jax.ops.segment_max(e, dst, num_segments=N)
    ex = jnp.exp(e - m[dst])
    s = jax.ops.segment_sum(ex, dst, num_segments=N)
    alpha = ex / (s[dst] + 1e-16)
    w0_p = jnp.zeros((EP,), jnp.float32).at[pos].set(
        alpha[perm, 0], unique_indices=True)
    w1_p = jnp.zeros((EP,), jnp.float32).at[pos].set(
        alpha[perm, 1], unique_indices=True)
    out1 = sc_agg(h.reshape(N, D), srcs_p, dls_p, w0_p, w1_p, s8)[:N]
    h1 = out1 + b1
    h1 = _batch_norm(h1, g1, be1)
    h1 = jax.nn.relu(h1)

    h2 = (h1 @ W2).reshape(N, 1, HID)
    alpha_s2 = jnp.sum(h2 * att_src2[None, :, :], axis=-1)
    alpha_d2 = jnp.sum(h2 * att_dst2[None, :, :], axis=-1)
    e2 = jax.nn.leaky_relu(alpha_s2[src] + alpha_d2[dst], negative_slope=0.2)
    m2 = jax.ops.segment_max(e2, dst, num_segments=N)
    ex2 = jnp.exp(e2 - m2[dst])
    s2 = jax.ops.segment_sum(ex2, dst, num_segments=N)
    alpha2 = ex2 / (s2[dst] + 1e-16)
    out2 = jax.ops.segment_sum(
        alpha2[:, :, None] * h2[src], dst, num_segments=N)
    h2o = out2.mean(axis=1) + b2
    h2b = _batch_norm(h2o, g2, be2)
    return h2b.mean(axis=0, keepdims=True)
